# baseline (speedup 1.0000x reference)
import math

import jax
import jax.numpy as jnp
from jax.experimental import pallas as pl
from jax.experimental.pallas import tpu as pltpu

N_NODES = 256
D = 128
ROWS = 64


def _edge_attn_kernel(x_ref, xr_ref, wq_ref, wk_ref, adj_ref, out_ref, m_ref):
    b = pl.program_id(0)
    ib = pl.program_id(1)

    @pl.when(jnp.logical_and(b == 0, ib == 0))
    def _():
        m = jnp.dot(wq_ref[:].T, wk_ref[:], preferred_element_type=jnp.float32)
        m_ref[:] = m * (1.0 / math.sqrt(D))

    xm = jnp.dot(xr_ref[0], m_ref[:], preferred_element_type=jnp.float32)
    logits = jnp.dot(xm, x_ref[0].T, preferred_element_type=jnp.float32)
    e = jnp.exp(logits - jnp.max(logits, axis=-1, keepdims=True))
    acc = jax.lax.dot_general(
        e, adj_ref[0],
        dimension_numbers=(((1,), (1,)), ((0,), (0,))),
        preferred_element_type=jnp.float32,
    )
    out_ref[0] = acc / jnp.sum(e, axis=-1, keepdims=True)


@jax.jit
def kernel(x, adj_matrix, W_q, W_k):
    B = adj_matrix.shape[0]
    xg = x.reshape(B, N_NODES, D)
    grid = (B, N_NODES // ROWS)
    out = pl.pallas_call(
        _edge_attn_kernel,
        grid=grid,
        in_specs=[
            pl.BlockSpec((1, N_NODES, D), lambda b, i: (b, 0, 0)),
            pl.BlockSpec((1, ROWS, D), lambda b, i: (b, i, 0)),
            pl.BlockSpec((D, D), lambda b, i: (0, 0)),
            pl.BlockSpec((D, D), lambda b, i: (0, 0)),
            pl.BlockSpec((1, ROWS, N_NODES, D), lambda b, i: (b, i, 0, 0)),
        ],
        out_specs=pl.BlockSpec((1, ROWS, D), lambda b, i: (b, i, 0)),
        out_shape=jax.ShapeDtypeStruct((B, N_NODES, D), jnp.float32),
        scratch_shapes=[pltpu.VMEM((D, D), jnp.float32)],
    )(xg, xg, W_q, W_k, adj_matrix)
    return out


# speedup vs baseline: 1.0046x; 1.0046x over previous
import math

import jax
import jax.numpy as jnp
from jax.experimental import pallas as pl
from jax.experimental.pallas import tpu as pltpu

N_NODES = 256
D = 128
ROWS = 64


def _edge_attn_kernel(x_ref, wq_ref, wk_ref, adj_ref, out_ref, m_ref):
    b = pl.program_id(0)
    ib = pl.program_id(1)

    @pl.when(jnp.logical_and(b == 0, ib == 0))
    def _():
        m = jnp.dot(wq_ref[:].T, wk_ref[:], preferred_element_type=jnp.float32)
        m_ref[:] = m * (1.0 / math.sqrt(D))

    x_rows = x_ref[0, pl.ds(ib * ROWS, ROWS), :]
    xm = jnp.dot(x_rows, m_ref[:], preferred_element_type=jnp.float32)
    logits = jnp.dot(xm, x_ref[0].T, preferred_element_type=jnp.float32)
    e = jnp.exp(logits - jnp.max(logits, axis=-1, keepdims=True))
    acc = jax.lax.dot_general(
        e, adj_ref[0],
        dimension_numbers=(((1,), (1,)), ((0,), (0,))),
        preferred_element_type=jnp.float32,
    )
    out_ref[0] = acc / jnp.sum(e, axis=-1, keepdims=True)


@jax.jit
def kernel(x, adj_matrix, W_q, W_k):
    B = adj_matrix.shape[0]
    xg = x.reshape(B, N_NODES, D)
    grid = (B, N_NODES // ROWS)
    out = pl.pallas_call(
        _edge_attn_kernel,
        grid=grid,
        in_specs=[
            pl.BlockSpec((1, N_NODES, D), lambda b, i: (b, 0, 0)),
            pl.BlockSpec((D, D), lambda b, i: (0, 0)),
            pl.BlockSpec((D, D), lambda b, i: (0, 0)),
            pl.BlockSpec((1, ROWS, N_NODES, D), lambda b, i: (b, i, 0, 0)),
        ],
        out_specs=pl.BlockSpec((1, ROWS, D), lambda b, i: (b, i, 0)),
        out_shape=jax.ShapeDtypeStruct((B, N_NODES, D), jnp.float32),
        scratch_shapes=[pltpu.VMEM((D, D), jnp.float32)],
        compiler_params=pltpu.CompilerParams(
            dimension_semantics=("arbitrary", "arbitrary"),
        ),
    )(xg, W_q, W_k, adj_matrix)
    return out
